# baseline (speedup 1.0000x reference)
import functools

import jax
import jax.numpy as jnp
from jax import lax
from jax.experimental import pallas as pl
from jax.experimental.pallas import tpu as pltpu
from jax.experimental.pallas import tpu_sc as plsc

NN = 50000
NP = 50176
R = 1024
G = NP // R
EREAL = 800000
EP = 802816
B = 128
NC, NS = 2, 16
ACC_R = 50176
STRIPE = ACC_R // NS
H = 128

@functools.lru_cache(maxsize=None)
def _make_sc_spmm(wacc, c_chunks, split, gather, tab_w):
    mesh = plsc.VectorSubcoreMesh(core_axis_name="c", subcore_axis_name="s",
                                  num_cores=NC, num_subcores=NS)
    ept = EP // (NC * NS) if split else EP // NS
    nsteps = ept // B
    cpc = 1 if split else c_chunks // NC
    n_out = NC if split else c_chunks
    K = 7 if nsteps % 7 == 0 else 4
    ngroups = nsteps // K

    scratch = [[pltpu.VMEM((2, B), jnp.int32) for _ in range(K)],
               [pltpu.VMEM((B, wacc), jnp.float32) for _ in range(K)],
               pltpu.VMEM_SHARED((ACC_R, wacc), jnp.float32),
               pltpu.SemaphoreType.DMA]

    def body(*refs):
        if gather:
            table_h, idx_h, zeros_h, out_h, idxbufs, rowbufs, acc, sem = refs
        else:
            ones_h, idx_h, zeros_h, out_h, idxbufs, rowbufs, acc, sem = refs
        ci = lax.axis_index("c")
        si = lax.axis_index("s")
        bb = ((ci * NS + si) if split else si) * nsteps
        if not gather:
            for b in range(K):
                pltpu.sync_copy(ones_h, rowbufs[b])
        for q in range(cpc):
            chunk = ci + NC * q if not split else None
            outidx = ci if split else chunk
            pltpu.sync_copy(zeros_h, acc.at[pl.ds(si * STRIPE, STRIPE)])
            plsc.subcore_barrier()

            if gather and tab_w != wacc:
                chunk_off = jnp.broadcast_to(chunk * NP, (16,)).astype(jnp.int32)

            def fire(g, b):
                idxb, rowb = idxbufs[b], rowbufs[b]
                pltpu.sync_copy(idx_h.at[bb + g * K + b], idxb)
                if gather:
                    if tab_w != wacc:
                        for r_ in range(B // 16):
                            sl = pl.ds(r_ * 16, 16)
                            idxb[0, sl] = idxb[0, sl] + chunk_off
                    pltpu.async_copy(table_h.at[idxb.at[0]], rowb, sem)

            for b in range(K):
                fire(0, b)

            def grp(g, carry):
                for b in range(K):
                    idxb, rowb = idxbufs[b], rowbufs[b]
                    if gather:
                        pltpu.make_async_copy(
                            table_h.at[idxb.at[0]], rowb, sem).wait()
                    pltpu.sync_copy(rowb, acc.at[idxb.at[1]], add=True)

                    @pl.when(g + 1 < ngroups)
                    def _():
                        fire(g + 1, b)
                return carry

            lax.fori_loop(0, ngroups, grp, 0)
            plsc.subcore_barrier()
            pltpu.sync_copy(
                acc.at[pl.ds(si * STRIPE, STRIPE)],
                out_h.at[outidx, pl.ds(si * STRIPE, STRIPE)])

    return pl.kernel(
        body,
        out_type=jax.ShapeDtypeStruct((n_out, ACC_R, wacc), jnp.float32),
        mesh=mesh,
        scratch_types=scratch,
        compiler_params=pltpu.CompilerParams(use_tc_tiling_on_sc=False),
    )


def _blk(w):
    return pl.BlockSpec((R, w), lambda i: (i, 0))


def _rep(w):
    return pl.BlockSpec((1, w), lambda i: (0, 0))


def _full(a, b):
    return pl.BlockSpec((a, b), lambda i: (0, 0))


def _cblk(k, w):
    return pl.BlockSpec((k, R, w), lambda i: (0, i, 0))


def _chunk_out(out_ref, t):
    for c in range(4):
        out_ref[c] = t[:, 32 * c:32 * c + 32]


def _prep_body(dacc_ref, x_ref, w_ref, dinv_ref, tbl_ref):
    d = dacc_ref[0][:, 0:1] + dacc_ref[1][:, 0:1] + 1.0
    di = lax.rsqrt(d)
    dinv_ref[...] = di
    hw = jnp.dot(x_ref[...], w_ref[...], preferred_element_type=jnp.float32)
    _chunk_out(tbl_ref, di * hw)


_prep = pl.pallas_call(
    _prep_body,
    grid=(G,),
    in_specs=[_cblk(2, 16), _blk(2), _full(2, H)],
    out_specs=[_blk(1), _cblk(4, 32)],
    out_shape=[jax.ShapeDtypeStruct((NP, 1), jnp.float32),
               jax.ShapeDtypeStruct((4, NP, 32), jnp.float32)],
)


def _stats_accum(i, z, sum_ref, sq_ref):
    rid = lax.broadcasted_iota(jnp.int32, (R, 1), 0) + i * R
    zm = jnp.where(rid < NN, z, 0.0)
    ps = jnp.sum(zm, axis=0, keepdims=True)
    psq = jnp.sum(zm * zm, axis=0, keepdims=True)

    @pl.when(i == 0)
    def _():
        sum_ref[...] = ps
        sq_ref[...] = psq

    @pl.when(i > 0)
    def _():
        sum_ref[...] += ps
        sq_ref[...] += psq


def _densein_body(agg_ref, tbl_ref, dinv_ref, b_ref, z_ref, sum_ref, sq_ref):
    a = jnp.concatenate([agg_ref[c] for c in range(4)], axis=-1)
    t = jnp.concatenate([tbl_ref[c] for c in range(4)], axis=-1)
    z = (a + t) * dinv_ref[...] + b_ref[...]
    z_ref[...] = z
    _stats_accum(pl.program_id(0), z, sum_ref, sq_ref)


_densein = pl.pallas_call(
    _densein_body,
    grid=(G,),
    in_specs=[_cblk(4, 32), _cblk(4, 32), _blk(1), _rep(H)],
    out_specs=[_blk(H), _rep(H), _rep(H)],
    out_shape=[jax.ShapeDtypeStruct((NP, H), jnp.float32),
               jax.ShapeDtypeStruct((1, H), jnp.float32),
               jax.ShapeDtypeStruct((1, H), jnp.float32)],
)


def _bn_core(z_ref, sum_ref, sq_ref, g_ref, bt_ref):
    m = sum_ref[...] * (1.0 / NN)
    v = sq_ref[...] * (1.0 / NN) - m * m
    inv = lax.rsqrt(v + 1e-5)
    return jnp.maximum((z_ref[...] - m) * inv * g_ref[...] + bt_ref[...], 0.0)


def _bnapply_body(z_ref, sum_ref, sq_ref, g_ref, bt_ref, dinv_ref, w_ref, out_ref):
    y = _bn_core(z_ref, sum_ref, sq_ref, g_ref, bt_ref)
    hw = jnp.dot(y, w_ref[...], preferred_element_type=jnp.float32)
    _chunk_out(out_ref, hw * dinv_ref[...])


_bnapply = pl.pallas_call(
    _bnapply_body,
    grid=(G,),
    in_specs=[_blk(H), _rep(H), _rep(H), _rep(H), _rep(H), _blk(1), _full(H, H)],
    out_specs=_cblk(4, 32),
    out_shape=jax.ShapeDtypeStruct((4, NP, 32), jnp.float32),
)


def _bnmm_body(z_ref, sum_ref, sq_ref, g_ref, bt_ref, w_ref, b_ref,
               z1_ref, sum1_ref, sq1_ref):
    y = _bn_core(z_ref, sum_ref, sq_ref, g_ref, bt_ref)
    z1 = jnp.dot(y, w_ref[...], preferred_element_type=jnp.float32) + b_ref[...]
    z1_ref[...] = z1
    _stats_accum(pl.program_id(0), z1, sum1_ref, sq1_ref)


_bnmm = pl.pallas_call(
    _bnmm_body,
    grid=(G,),
    in_specs=[_blk(H), _rep(H), _rep(H), _rep(H), _rep(H), _full(H, 32), _rep(32)],
    out_specs=[_blk(32), _rep(32), _rep(32)],
    out_shape=[jax.ShapeDtypeStruct((NP, 32), jnp.float32),
               jax.ShapeDtypeStruct((1, 32), jnp.float32),
               jax.ShapeDtypeStruct((1, 32), jnp.float32)],
)


def _final_body(z_ref, sum_ref, sq_ref, g_ref, bt_ref, w_ref, b_ref, o_ref):
    y = _bn_core(z_ref, sum_ref, sq_ref, g_ref, bt_ref)
    o_ref[...] = jnp.tanh(
        jnp.dot(y, w_ref[...], preferred_element_type=jnp.float32) + b_ref[...])


_final = pl.pallas_call(
    _final_body,
    grid=(G,),
    in_specs=[_blk(32), _rep(32), _rep(32), _rep(32), _rep(32), _full(32, 2), _rep(2)],
    out_specs=_blk(2),
    out_shape=jax.ShapeDtypeStruct((NP, 2), jnp.float32),
)


def kernel(x, edge_index, W1, b1, gamma1, beta1, W2, b2, gamma2, beta2,
           W3, b3, gamma3, beta3, fc1_W, fc1_b, gamma_fc, beta_fc, fc2_W, fc2_b):
    f32 = jnp.float32
    _deg_sc = _make_sc_spmm(wacc=16, c_chunks=1, split=True, gather=False, tab_w=16)
    _spmm_sc = _make_sc_spmm(wacc=32, c_chunks=4, split=False, gather=True, tab_w=H)
    pad = jnp.full((EP - EREAL,), NN, jnp.int32)
    src = jnp.concatenate([edge_index[0].astype(jnp.int32), pad])
    dst = jnp.concatenate([edge_index[1].astype(jnp.int32), pad])
    idx3 = jnp.stack([src.reshape(-1, B), dst.reshape(-1, B)], axis=1)
    x_pad = jnp.zeros((NP, 2), f32).at[:NN].set(x)
    ones16 = jnp.ones((B, 16), f32)
    z16 = jnp.zeros((STRIPE, 16), f32)
    z32 = jnp.zeros((STRIPE, 32), f32)

    degacc = _deg_sc(ones16, idx3, z16)
    dinv, tbl = _prep(degacc, x_pad, W1)
    for (bi, gi, bti, Wn) in ((b1, gamma1, beta1, W2), (b2, gamma2, beta2, W3)):
        agg = _spmm_sc(tbl.reshape(4 * NP, 32), idx3, z32)
        z, s, q = _densein(agg, tbl, dinv, bi.reshape(1, H))
        tbl = _bnapply(z, s, q, gi.reshape(1, H), bti.reshape(1, H), dinv, Wn)
    agg = _spmm_sc(tbl.reshape(4 * NP, 32), idx3, z32)
    z3, s3, q3 = _densein(agg, tbl, dinv, b3.reshape(1, H))
    zf, sf, qf = _bnmm(z3, s3, q3, gamma3.reshape(1, H), beta3.reshape(1, H),
                       fc1_W, fc1_b.reshape(1, 32))
    out = _final(zf, sf, qf, gamma_fc.reshape(1, 32), beta_fc.reshape(1, 32),
                 fc2_W, fc2_b.reshape(1, 2))
    return out[:NN]


# speedup vs baseline: 1.0013x; 1.0013x over previous
import functools

import jax
import jax.numpy as jnp
from jax import lax
from jax.experimental import pallas as pl
from jax.experimental.pallas import tpu as pltpu
from jax.experimental.pallas import tpu_sc as plsc

NN = 50000
NP = 50176
R = 1024
G = NP // R
EREAL = 800000
EP = 802816
B = 128
NC, NS = 2, 16
ACC_R = 50176
STRIPE = ACC_R // NS
H = 128

@functools.lru_cache(maxsize=None)
def _make_sc_spmm(wacc, c_chunks, split, gather, tab_w):
    mesh = plsc.VectorSubcoreMesh(core_axis_name="c", subcore_axis_name="s",
                                  num_cores=NC, num_subcores=NS)
    ept = EP // (NC * NS) if split else EP // NS
    nsteps = ept // B
    cpc = 1 if split else c_chunks // NC
    n_out = NC if split else c_chunks
    K = 4
    ngroups = nsteps // K

    scratch = [[pltpu.VMEM((2, B), jnp.int32) for _ in range(K)],
               [pltpu.VMEM((B, wacc), jnp.float32) for _ in range(K)],
               pltpu.VMEM_SHARED((ACC_R, wacc), jnp.float32),
               pltpu.SemaphoreType.DMA]

    def body(*refs):
        if gather:
            table_h, idx_h, zeros_h, out_h, idxbufs, rowbufs, acc, sem = refs
        else:
            ones_h, idx_h, zeros_h, out_h, idxbufs, rowbufs, acc, sem = refs
        ci = lax.axis_index("c")
        si = lax.axis_index("s")
        bb = ((ci * NS + si) if split else si) * nsteps
        if not gather:
            for b in range(K):
                pltpu.sync_copy(ones_h, rowbufs[b])
        for q in range(cpc):
            chunk = ci + NC * q if not split else None
            outidx = ci if split else chunk
            pltpu.sync_copy(zeros_h, acc.at[pl.ds(si * STRIPE, STRIPE)])
            plsc.subcore_barrier()

            if gather and tab_w != wacc:
                chunk_off = jnp.broadcast_to(chunk * NP, (16,)).astype(jnp.int32)

            def fire(g, b):
                idxb, rowb = idxbufs[b], rowbufs[b]
                pltpu.sync_copy(idx_h.at[bb + g * K + b], idxb)
                if gather:
                    if tab_w != wacc:
                        for r_ in range(B // 16):
                            sl = pl.ds(r_ * 16, 16)
                            idxb[0, sl] = idxb[0, sl] + chunk_off
                    pltpu.async_copy(table_h.at[idxb.at[0]], rowb, sem)

            for b in range(K):
                fire(0, b)

            def grp(g, carry):
                for b in range(K):
                    idxb, rowb = idxbufs[b], rowbufs[b]
                    if gather:
                        pltpu.make_async_copy(
                            table_h.at[idxb.at[0]], rowb, sem).wait()
                    pltpu.sync_copy(rowb, acc.at[idxb.at[1]], add=True)

                    @pl.when(g + 1 < ngroups)
                    def _():
                        fire(g + 1, b)
                return carry

            lax.fori_loop(0, ngroups, grp, 0)
            plsc.subcore_barrier()
            pltpu.sync_copy(
                acc.at[pl.ds(si * STRIPE, STRIPE)],
                out_h.at[outidx, pl.ds(si * STRIPE, STRIPE)])

    return pl.kernel(
        body,
        out_type=jax.ShapeDtypeStruct((n_out, ACC_R, wacc), jnp.float32),
        mesh=mesh,
        scratch_types=scratch,
        compiler_params=pltpu.CompilerParams(use_tc_tiling_on_sc=False),
    )


def _blk(w):
    return pl.BlockSpec((R, w), lambda i: (i, 0))


def _rep(w):
    return pl.BlockSpec((1, w), lambda i: (0, 0))


def _full(a, b):
    return pl.BlockSpec((a, b), lambda i: (0, 0))


def _cblk(k, w):
    return pl.BlockSpec((k, R, w), lambda i: (0, i, 0))


def _chunk_out(out_ref, t):
    for c in range(4):
        out_ref[c] = t[:, 32 * c:32 * c + 32]


def _prep_body(dacc_ref, x_ref, w_ref, dinv_ref, tbl_ref):
    d = dacc_ref[0][:, 0:1] + dacc_ref[1][:, 0:1] + 1.0
    di = lax.rsqrt(d)
    dinv_ref[...] = di
    hw = jnp.dot(x_ref[...], w_ref[...], preferred_element_type=jnp.float32)
    _chunk_out(tbl_ref, di * hw)


_prep = pl.pallas_call(
    _prep_body,
    grid=(G,),
    in_specs=[_cblk(2, 16), _blk(2), _full(2, H)],
    out_specs=[_blk(1), _cblk(4, 32)],
    out_shape=[jax.ShapeDtypeStruct((NP, 1), jnp.float32),
               jax.ShapeDtypeStruct((4, NP, 32), jnp.float32)],
)


def _stats_accum(i, z, sum_ref, sq_ref):
    rid = lax.broadcasted_iota(jnp.int32, (R, 1), 0) + i * R
    zm = jnp.where(rid < NN, z, 0.0)
    ps = jnp.sum(zm, axis=0, keepdims=True)
    psq = jnp.sum(zm * zm, axis=0, keepdims=True)

    @pl.when(i == 0)
    def _():
        sum_ref[...] = ps
        sq_ref[...] = psq

    @pl.when(i > 0)
    def _():
        sum_ref[...] += ps
        sq_ref[...] += psq


def _densein_body(agg_ref, tbl_ref, dinv_ref, b_ref, z_ref, sum_ref, sq_ref):
    a = jnp.concatenate([agg_ref[c] for c in range(4)], axis=-1)
    t = jnp.concatenate([tbl_ref[c] for c in range(4)], axis=-1)
    z = (a + t) * dinv_ref[...] + b_ref[...]
    z_ref[...] = z
    _stats_accum(pl.program_id(0), z, sum_ref, sq_ref)


_densein = pl.pallas_call(
    _densein_body,
    grid=(G,),
    in_specs=[_cblk(4, 32), _cblk(4, 32), _blk(1), _rep(H)],
    out_specs=[_blk(H), _rep(H), _rep(H)],
    out_shape=[jax.ShapeDtypeStruct((NP, H), jnp.float32),
               jax.ShapeDtypeStruct((1, H), jnp.float32),
               jax.ShapeDtypeStruct((1, H), jnp.float32)],
)


def _bn_core(z_ref, sum_ref, sq_ref, g_ref, bt_ref):
    m = sum_ref[...] * (1.0 / NN)
    v = sq_ref[...] * (1.0 / NN) - m * m
    inv = lax.rsqrt(v + 1e-5)
    return jnp.maximum((z_ref[...] - m) * inv * g_ref[...] + bt_ref[...], 0.0)


def _bnapply_body(z_ref, sum_ref, sq_ref, g_ref, bt_ref, dinv_ref, w_ref, out_ref):
    y = _bn_core(z_ref, sum_ref, sq_ref, g_ref, bt_ref)
    hw = jnp.dot(y, w_ref[...], preferred_element_type=jnp.float32)
    _chunk_out(out_ref, hw * dinv_ref[...])


_bnapply = pl.pallas_call(
    _bnapply_body,
    grid=(G,),
    in_specs=[_blk(H), _rep(H), _rep(H), _rep(H), _rep(H), _blk(1), _full(H, H)],
    out_specs=_cblk(4, 32),
    out_shape=jax.ShapeDtypeStruct((4, NP, 32), jnp.float32),
)


def _bnmm_body(z_ref, sum_ref, sq_ref, g_ref, bt_ref, w_ref, b_ref,
               z1_ref, sum1_ref, sq1_ref):
    y = _bn_core(z_ref, sum_ref, sq_ref, g_ref, bt_ref)
    z1 = jnp.dot(y, w_ref[...], preferred_element_type=jnp.float32) + b_ref[...]
    z1_ref[...] = z1
    _stats_accum(pl.program_id(0), z1, sum1_ref, sq1_ref)


_bnmm = pl.pallas_call(
    _bnmm_body,
    grid=(G,),
    in_specs=[_blk(H), _rep(H), _rep(H), _rep(H), _rep(H), _full(H, 32), _rep(32)],
    out_specs=[_blk(32), _rep(32), _rep(32)],
    out_shape=[jax.ShapeDtypeStruct((NP, 32), jnp.float32),
               jax.ShapeDtypeStruct((1, 32), jnp.float32),
               jax.ShapeDtypeStruct((1, 32), jnp.float32)],
)


def _final_body(z_ref, sum_ref, sq_ref, g_ref, bt_ref, w_ref, b_ref, o_ref):
    y = _bn_core(z_ref, sum_ref, sq_ref, g_ref, bt_ref)
    o_ref[...] = jnp.tanh(
        jnp.dot(y, w_ref[...], preferred_element_type=jnp.float32) + b_ref[...])


_final = pl.pallas_call(
    _final_body,
    grid=(G,),
    in_specs=[_blk(32), _rep(32), _rep(32), _rep(32), _rep(32), _full(32, 2), _rep(2)],
    out_specs=_blk(2),
    out_shape=jax.ShapeDtypeStruct((NP, 2), jnp.float32),
)


def kernel(x, edge_index, W1, b1, gamma1, beta1, W2, b2, gamma2, beta2,
           W3, b3, gamma3, beta3, fc1_W, fc1_b, gamma_fc, beta_fc, fc2_W, fc2_b):
    f32 = jnp.float32
    _deg_sc = _make_sc_spmm(wacc=16, c_chunks=1, split=True, gather=False, tab_w=16)
    _spmm_sc = _make_sc_spmm(wacc=32, c_chunks=4, split=False, gather=True, tab_w=H)
    pad = jnp.full((EP - EREAL,), NN, jnp.int32)
    src = jnp.concatenate([edge_index[0].astype(jnp.int32), pad])
    dst = jnp.concatenate([edge_index[1].astype(jnp.int32), pad])
    idx3 = jnp.stack([src.reshape(-1, B), dst.reshape(-1, B)], axis=1)
    x_pad = jnp.zeros((NP, 2), f32).at[:NN].set(x)
    ones16 = jnp.ones((B, 16), f32)
    z16 = jnp.zeros((STRIPE, 16), f32)
    z32 = jnp.zeros((STRIPE, 32), f32)

    degacc = _deg_sc(ones16, idx3, z16)
    dinv, tbl = _prep(degacc, x_pad, W1)
    for (bi, gi, bti, Wn) in ((b1, gamma1, beta1, W2), (b2, gamma2, beta2, W3)):
        agg = _spmm_sc(tbl.reshape(4 * NP, 32), idx3, z32)
        z, s, q = _densein(agg, tbl, dinv, bi.reshape(1, H))
        tbl = _bnapply(z, s, q, gi.reshape(1, H), bti.reshape(1, H), dinv, Wn)
    agg = _spmm_sc(tbl.reshape(4 * NP, 32), idx3, z32)
    z3, s3, q3 = _densein(agg, tbl, dinv, b3.reshape(1, H))
    zf, sf, qf = _bnmm(z3, s3, q3, gamma3.reshape(1, H), beta3.reshape(1, H),
                       fc1_W, fc1_b.reshape(1, 32))
    out = _final(zf, sf, qf, gamma_fc.reshape(1, 32), beta_fc.reshape(1, 32),
                 fc2_W, fc2_b.reshape(1, 2))
    return out[:NN]


# speedup vs baseline: 1.2901x; 1.2884x over previous
import functools

import jax
import jax.numpy as jnp
from jax import lax
from jax.experimental import pallas as pl
from jax.experimental.pallas import tpu as pltpu
from jax.experimental.pallas import tpu_sc as plsc

NN = 50000
NP = 50176
R = 1024
G = NP // R
EREAL = 800000
EP = 802816
B = 128
NC, NS = 2, 16
ACC_R = 50176
STRIPE = ACC_R // NS
H = 128

@functools.lru_cache(maxsize=None)
def _make_sc_spmm(wacc, c_chunks, split, gather, tab_w):
    mesh = plsc.VectorSubcoreMesh(core_axis_name="c", subcore_axis_name="s",
                                  num_cores=NC, num_subcores=NS)
    ept = EP // (NC * NS) if split else EP // NS
    nsteps = ept // B
    cpc = 1 if split else c_chunks // NC
    n_out = NC if split else c_chunks
    K = 4
    ngroups = nsteps // K

    if gather:
        idx_scratch = [pltpu.VMEM((K, 2, B), jnp.int32) for _ in range(2)]
    else:
        idx_scratch = [pltpu.VMEM((2, B), jnp.int32) for _ in range(K)]
    scratch = [idx_scratch,
               [pltpu.VMEM((B, wacc), jnp.float32) for _ in range(K)],
               pltpu.VMEM_SHARED((ACC_R, wacc), jnp.float32),
               pltpu.SemaphoreType.DMA]

    def body(*refs):
        if gather:
            table_h, idx_h, zeros_h, out_h, idxbufs, rowbufs, acc, sem = refs
        else:
            ones_h, idx_h, zeros_h, out_h, idxbufs, rowbufs, acc, sem = refs
        ci = lax.axis_index("c")
        si = lax.axis_index("s")
        bb = ((ci * NS + si) if split else si) * nsteps
        if not gather:
            for b in range(K):
                pltpu.sync_copy(ones_h, rowbufs[b])
        for q in range(cpc):
            chunk = ci + NC * q if not split else None
            outidx = ci if split else chunk
            pltpu.sync_copy(zeros_h, acc.at[pl.ds(si * STRIPE, STRIPE)])
            plsc.subcore_barrier()

            if gather:
                chunk_off = jnp.broadcast_to(chunk * NP, (16,)).astype(jnp.int32)

                def load_group(g, P):
                    pltpu.sync_copy(idx_h.at[pl.ds(bb + g * K, K)], P)
                    if tab_w != wacc:
                        for b_ in range(K):
                            for r_ in range(B // 16):
                                sl = pl.ds(r_ * 16, 16)
                                P[b_, 0, sl] = P[b_, 0, sl] + chunk_off

                load_group(0, idxbufs[0])
                for b in range(K):
                    pltpu.async_copy(
                        table_h.at[idxbufs[0].at[b, 0]], rowbufs[b], sem)
                load_group(1, idxbufs[1])

                def handle(g, P, Q):
                    for b in range(K):
                        rowb = rowbufs[b]
                        pltpu.make_async_copy(
                            table_h.at[P.at[b, 0]], rowb, sem).wait()
                        pltpu.sync_copy(rowb, acc.at[P.at[b, 1]], add=True)

                        @pl.when(g + 1 < ngroups)
                        def _():
                            pltpu.async_copy(
                                table_h.at[Q.at[b, 0]], rowb, sem)

                    @pl.when(g + 2 < ngroups)
                    def _():
                        load_group(g + 2, P)

                def grp2(gg, carry):
                    handle(2 * gg, idxbufs[0], idxbufs[1])
                    handle(2 * gg + 1, idxbufs[1], idxbufs[0])
                    return carry

                lax.fori_loop(0, ngroups // 2, grp2, 0)
            else:
                def grp(g, carry):
                    for b in range(K):
                        idxb, rowb = idxbufs[b], rowbufs[b]
                        pltpu.sync_copy(idx_h.at[bb + g * K + b], idxb)
                        pltpu.sync_copy(rowb, acc.at[idxb.at[1]], add=True)
                    return carry

                lax.fori_loop(0, ngroups, grp, 0)
            plsc.subcore_barrier()
            pltpu.sync_copy(
                acc.at[pl.ds(si * STRIPE, STRIPE)],
                out_h.at[outidx, pl.ds(si * STRIPE, STRIPE)])

    return pl.kernel(
        body,
        out_type=jax.ShapeDtypeStruct((n_out, ACC_R, wacc), jnp.float32),
        mesh=mesh,
        scratch_types=scratch,
        compiler_params=pltpu.CompilerParams(use_tc_tiling_on_sc=False),
    )


def _blk(w):
    return pl.BlockSpec((R, w), lambda i: (i, 0))


def _rep(w):
    return pl.BlockSpec((1, w), lambda i: (0, 0))


def _full(a, b):
    return pl.BlockSpec((a, b), lambda i: (0, 0))


def _cblk(k, w):
    return pl.BlockSpec((k, R, w), lambda i: (0, i, 0))


def _chunk_out(out_ref, t):
    for c in range(4):
        out_ref[c] = t[:, 32 * c:32 * c + 32]


def _prep_body(dacc_ref, x_ref, w_ref, dinv_ref, tbl_ref):
    d = dacc_ref[0][:, 0:1] + dacc_ref[1][:, 0:1] + 1.0
    di = lax.rsqrt(d)
    dinv_ref[...] = di
    hw = jnp.dot(x_ref[...], w_ref[...], preferred_element_type=jnp.float32)
    _chunk_out(tbl_ref, di * hw)


_prep = pl.pallas_call(
    _prep_body,
    grid=(G,),
    in_specs=[_cblk(2, 16), _blk(2), _full(2, H)],
    out_specs=[_blk(1), _cblk(4, 32)],
    out_shape=[jax.ShapeDtypeStruct((NP, 1), jnp.float32),
               jax.ShapeDtypeStruct((4, NP, 32), jnp.float32)],
)


def _stats_accum(i, z, sum_ref, sq_ref):
    rid = lax.broadcasted_iota(jnp.int32, (R, 1), 0) + i * R
    zm = jnp.where(rid < NN, z, 0.0)
    ps = jnp.sum(zm, axis=0, keepdims=True)
    psq = jnp.sum(zm * zm, axis=0, keepdims=True)

    @pl.when(i == 0)
    def _():
        sum_ref[...] = ps
        sq_ref[...] = psq

    @pl.when(i > 0)
    def _():
        sum_ref[...] += ps
        sq_ref[...] += psq


def _densein_body(agg_ref, tbl_ref, dinv_ref, b_ref, z_ref, sum_ref, sq_ref):
    a = jnp.concatenate([agg_ref[c] for c in range(4)], axis=-1)
    t = jnp.concatenate([tbl_ref[c] for c in range(4)], axis=-1)
    z = (a + t) * dinv_ref[...] + b_ref[...]
    z_ref[...] = z
    _stats_accum(pl.program_id(0), z, sum_ref, sq_ref)


_densein = pl.pallas_call(
    _densein_body,
    grid=(G,),
    in_specs=[_cblk(4, 32), _cblk(4, 32), _blk(1), _rep(H)],
    out_specs=[_blk(H), _rep(H), _rep(H)],
    out_shape=[jax.ShapeDtypeStruct((NP, H), jnp.float32),
               jax.ShapeDtypeStruct((1, H), jnp.float32),
               jax.ShapeDtypeStruct((1, H), jnp.float32)],
)


def _bn_core(z_ref, sum_ref, sq_ref, g_ref, bt_ref):
    m = sum_ref[...] * (1.0 / NN)
    v = sq_ref[...] * (1.0 / NN) - m * m
    inv = lax.rsqrt(v + 1e-5)
    return jnp.maximum((z_ref[...] - m) * inv * g_ref[...] + bt_ref[...], 0.0)


def _bnapply_body(z_ref, sum_ref, sq_ref, g_ref, bt_ref, dinv_ref, w_ref, out_ref):
    y = _bn_core(z_ref, sum_ref, sq_ref, g_ref, bt_ref)
    hw = jnp.dot(y, w_ref[...], preferred_element_type=jnp.float32)
    _chunk_out(out_ref, hw * dinv_ref[...])


_bnapply = pl.pallas_call(
    _bnapply_body,
    grid=(G,),
    in_specs=[_blk(H), _rep(H), _rep(H), _rep(H), _rep(H), _blk(1), _full(H, H)],
    out_specs=_cblk(4, 32),
    out_shape=jax.ShapeDtypeStruct((4, NP, 32), jnp.float32),
)


def _bnmm_body(z_ref, sum_ref, sq_ref, g_ref, bt_ref, w_ref, b_ref,
               z1_ref, sum1_ref, sq1_ref):
    y = _bn_core(z_ref, sum_ref, sq_ref, g_ref, bt_ref)
    z1 = jnp.dot(y, w_ref[...], preferred_element_type=jnp.float32) + b_ref[...]
    z1_ref[...] = z1
    _stats_accum(pl.program_id(0), z1, sum1_ref, sq1_ref)


_bnmm = pl.pallas_call(
    _bnmm_body,
    grid=(G,),
    in_specs=[_blk(H), _rep(H), _rep(H), _rep(H), _rep(H), _full(H, 32), _rep(32)],
    out_specs=[_blk(32), _rep(32), _rep(32)],
    out_shape=[jax.ShapeDtypeStruct((NP, 32), jnp.float32),
               jax.ShapeDtypeStruct((1, 32), jnp.float32),
               jax.ShapeDtypeStruct((1, 32), jnp.float32)],
)


def _final_body(z_ref, sum_ref, sq_ref, g_ref, bt_ref, w_ref, b_ref, o_ref):
    y = _bn_core(z_ref, sum_ref, sq_ref, g_ref, bt_ref)
    o_ref[...] = jnp.tanh(
        jnp.dot(y, w_ref[...], preferred_element_type=jnp.float32) + b_ref[...])


_final = pl.pallas_call(
    _final_body,
    grid=(G,),
    in_specs=[_blk(32), _rep(32), _rep(32), _rep(32), _rep(32), _full(32, 2), _rep(2)],
    out_specs=_blk(2),
    out_shape=jax.ShapeDtypeStruct((NP, 2), jnp.float32),
)


def kernel(x, edge_index, W1, b1, gamma1, beta1, W2, b2, gamma2, beta2,
           W3, b3, gamma3, beta3, fc1_W, fc1_b, gamma_fc, beta_fc, fc2_W, fc2_b):
    f32 = jnp.float32
    _deg_sc = _make_sc_spmm(wacc=16, c_chunks=1, split=True, gather=False, tab_w=16)
    _spmm_sc = _make_sc_spmm(wacc=32, c_chunks=4, split=False, gather=True, tab_w=H)
    pad = jnp.full((EP - EREAL,), NN, jnp.int32)
    src = jnp.concatenate([edge_index[0].astype(jnp.int32), pad])
    dst = jnp.concatenate([edge_index[1].astype(jnp.int32), pad])
    idx3 = jnp.stack([src.reshape(-1, B), dst.reshape(-1, B)], axis=1)
    x_pad = jnp.zeros((NP, 2), f32).at[:NN].set(x)
    ones16 = jnp.ones((B, 16), f32)
    z16 = jnp.zeros((STRIPE, 16), f32)
    z32 = jnp.zeros((STRIPE, 32), f32)

    degacc = _deg_sc(ones16, idx3, z16)
    dinv, tbl = _prep(degacc, x_pad, W1)
    for (bi, gi, bti, Wn) in ((b1, gamma1, beta1, W2), (b2, gamma2, beta2, W3)):
        agg = _spmm_sc(tbl.reshape(4 * NP, 32), idx3, z32)
        z, s, q = _densein(agg, tbl, dinv, bi.reshape(1, H))
        tbl = _bnapply(z, s, q, gi.reshape(1, H), bti.reshape(1, H), dinv, Wn)
    agg = _spmm_sc(tbl.reshape(4 * NP, 32), idx3, z32)
    z3, s3, q3 = _densein(agg, tbl, dinv, b3.reshape(1, H))
    zf, sf, qf = _bnmm(z3, s3, q3, gamma3.reshape(1, H), beta3.reshape(1, H),
                       fc1_W, fc1_b.reshape(1, 32))
    out = _final(zf, sf, qf, gamma_fc.reshape(1, 32), beta_fc.reshape(1, 32),
                 fc2_W, fc2_b.reshape(1, 2))
    return out[:NN]
